# Initial kernel scaffold; baseline (speedup 1.0000x reference)
import jax
import jax.numpy as jnp
from jax.experimental import pallas as pl


def kernel(x, edge_index, Ws1, Wn1, b1, Ws2, Wn2, b2):
    raise NotImplementedError("write your pallas kernel here")


# speedup vs baseline: 3.3744x; 3.3744x over previous
import functools

import jax
import jax.numpy as jnp
from jax import lax
from jax.experimental import pallas as pl
from jax.experimental.pallas import tpu as pltpu
from jax.experimental.pallas import tpu_sc as plsc

N = 10000
E = 320000
D = 128
NC = 2
NS = 16
NW = NC * NS
PERW = 10240
CH = 128
C = PERW // CH
EP = NW * PERW
NA = 10240
RPT = NA // NS

_mesh = plsc.VectorSubcoreMesh(core_axis_name="c", subcore_axis_name="s")


def _make_sc(with_deg):
    out_type = [jax.ShapeDtypeStruct((NC, NA, D), jnp.float32)]
    if with_deg:
        out_type.append(jax.ShapeDtypeStruct((NC, NA), jnp.float32))
    scratch = [
        pltpu.VMEM((C, CH), jnp.int32),
        pltpu.VMEM((C, CH), jnp.int32),
        pltpu.VMEM((CH, D), jnp.float32),
        pltpu.VMEM((CH,), jnp.float32),
        pltpu.MemorySpace.VMEM_SHARED((NA, D), jnp.float32),
        pltpu.MemorySpace.VMEM_SHARED((NA,), jnp.float32),
        pltpu.SemaphoreType.DMA,
    ]

    def body(h_hbm, src_hbm, dst_hbm, zb_hbm, zd_hbm, one_hbm, *rest):
        if with_deg:
            agg_out, deg_out = rest[0], rest[1]
            srcw, dstw, rowbuf, onesv, agg_sh, deg_sh, sem = rest[2:]
        else:
            agg_out = rest[0]
            srcw, dstw, rowbuf, onesv, agg_sh, deg_sh, sem = rest[1:]
        c = lax.axis_index("c")
        s = lax.axis_index("s")
        wid = c * NS + s
        pltpu.sync_copy(zb_hbm, agg_sh.at[pl.ds(s * RPT, RPT)])
        if with_deg:
            pltpu.sync_copy(zd_hbm, deg_sh.at[pl.ds(s * RPT, RPT)])
            pltpu.sync_copy(one_hbm, onesv)
        pltpu.sync_copy(src_hbm.at[wid], srcw)
        pltpu.sync_copy(dst_hbm.at[wid], dstw)
        plsc.subcore_barrier()

        def step(j, carry):
            pltpu.sync_copy(h_hbm.at[srcw.at[j]], rowbuf)
            pltpu.sync_copy(rowbuf, agg_sh.at[dstw.at[j]], add=True)
            if with_deg:
                pltpu.sync_copy(onesv, deg_sh.at[dstw.at[j]], add=True)
            return carry

        lax.fori_loop(0, C, step, 0)
        plsc.subcore_barrier()
        pltpu.sync_copy(agg_sh.at[pl.ds(s * RPT, RPT)],
                        agg_out.at[c, pl.ds(s * RPT, RPT)])
        if with_deg:
            pltpu.sync_copy(deg_sh.at[pl.ds(s * RPT, RPT)],
                            deg_out.at[c, pl.ds(s * RPT, RPT)])

    return pl.kernel(body, out_type=out_type, mesh=_mesh,
                     scratch_types=scratch)


_sc_deg = _make_sc(True)
_sc_nodeg = _make_sc(False)

BM = 1000


def _tc_body(h_ref, agg_ref, deg_ref, Ws_ref, Wn_ref, b_ref, out_ref):
    inv = 1.0 / jnp.maximum(deg_ref[0] + deg_ref[1], 1.0)
    mean = (agg_ref[0] + agg_ref[1]) * inv
    acc = jnp.dot(h_ref[...], Ws_ref[...], preferred_element_type=jnp.float32)
    acc += jnp.dot(mean, Wn_ref[...], preferred_element_type=jnp.float32)
    acc += b_ref[...]
    out_ref[...] = jnp.maximum(acc, 0.0)


def _tc_dense(h, aggP, degP, Ws, Wn, b):
    return pl.pallas_call(
        _tc_body,
        grid=(N // BM,),
        in_specs=[
            pl.BlockSpec((BM, D), lambda i: (i, 0)),
            pl.BlockSpec((NC, BM, D), lambda i: (0, i, 0)),
            pl.BlockSpec((NC, BM, 1), lambda i: (0, i, 0)),
            pl.BlockSpec((D, D), lambda i: (0, 0)),
            pl.BlockSpec((D, D), lambda i: (0, 0)),
            pl.BlockSpec((1, D), lambda i: (0, 0)),
        ],
        out_specs=pl.BlockSpec((BM, D), lambda i: (i, 0)),
        out_shape=jax.ShapeDtypeStruct((N, D), jnp.float32),
    )(h, aggP, degP, Ws, Wn, b.reshape(1, D))


def kernel(x, edge_index, Ws1, Wn1, b1, Ws2, Wn2, b2):
    src = edge_index[0]
    dst = edge_index[1]
    pad = EP - E
    srcp = jnp.concatenate(
        [src, jnp.zeros((pad,), jnp.int32)]).reshape(NW, C, CH)
    dstp = jnp.concatenate(
        [dst, jnp.full((pad,), N, jnp.int32)]).reshape(NW, C, CH)
    zb = jnp.zeros((RPT, D), jnp.float32)
    zd = jnp.zeros((RPT,), jnp.float32)
    one = jnp.ones((CH,), jnp.float32)

    aggP1, degP = _sc_deg(x, srcp, dstp, zb, zd, one)
    degP = degP[:, :N, None]
    h1 = _tc_dense(x, aggP1[:, :N], degP, Ws1, Wn1, b1)
    aggP2 = _sc_nodeg(h1, srcp, dstp, zb, zd, one)
    if isinstance(aggP2, (list, tuple)):
        aggP2 = aggP2[0]
    h2 = _tc_dense(h1, aggP2[:, :N], degP, Ws2, Wn2, b2)
    return h2


# speedup vs baseline: 3.9118x; 1.1593x over previous
import functools

import jax
import jax.numpy as jnp
from jax import lax
from jax.experimental import pallas as pl
from jax.experimental.pallas import tpu as pltpu
from jax.experimental.pallas import tpu_sc as plsc

N = 10000
E = 320000
D = 128
NC = 2
NS = 16
NW = NC * NS
PERW = 10240
CH = 128
C = PERW // CH
EP = NW * PERW
NA = 10240
RPT = NA // NS

_mesh = plsc.VectorSubcoreMesh(core_axis_name="c", subcore_axis_name="s")


def _make_sc(with_deg):
    out_type = [jax.ShapeDtypeStruct((NC, NA, D), jnp.float32)]
    if with_deg:
        out_type.append(jax.ShapeDtypeStruct((NC, NA), jnp.float32))
    scratch = [
        pltpu.VMEM((2, CH), jnp.int32),
        pltpu.VMEM((2, CH), jnp.int32),
        pltpu.VMEM((CH, D), jnp.float32),
        pltpu.VMEM((CH, D), jnp.float32),
        pltpu.VMEM((CH,), jnp.float32),
        pltpu.MemorySpace.VMEM_SHARED((NA, D), jnp.float32),
        pltpu.MemorySpace.VMEM_SHARED((NA,), jnp.float32),
        pltpu.SemaphoreType.DMA,
        pltpu.SemaphoreType.DMA,
        pltpu.SemaphoreType.DMA,
        pltpu.SemaphoreType.DMA,
    ]

    def body(h_hbm, idx_hbm, zb_hbm, zd_hbm, one_hbm, *rest):
        if with_deg:
            agg_out, deg_out = rest[0], rest[1]
            scr = rest[2:]
        else:
            agg_out = rest[0]
            scr = rest[1:]
        (ibuf0, ibuf1, buf0, buf1, onesv, agg_sh, deg_sh,
         sem0, sem1, semi0, semi1) = scr
        ibufs, bufs = (ibuf0, ibuf1), (buf0, buf1)
        sems, semis = (sem0, sem1), (semi0, semi1)
        c = lax.axis_index("c")
        s = lax.axis_index("s")
        wid = c * NS + s
        pltpu.sync_copy(zb_hbm, agg_sh.at[pl.ds(s * RPT, RPT)])
        if with_deg:
            pltpu.sync_copy(zd_hbm, deg_sh.at[pl.ds(s * RPT, RPT)])
            pltpu.sync_copy(one_hbm, onesv)
        plsc.subcore_barrier()

        pltpu.sync_copy(idx_hbm.at[wid, 0], ibuf0)
        pltpu.async_copy(idx_hbm.at[wid, 1], ibuf1, semi1)
        pltpu.async_copy(h_hbm.at[ibuf0.at[0]], buf0, sem0)

        def half(j, x, last):
            y = 1 - x
            ib, gb = ibufs[x], bufs[x]
            ibn, gbn = ibufs[y], bufs[y]

            @pl.when(j + 1 < C)
            def _():
                pltpu.make_async_copy(
                    idx_hbm.at[wid, j + 1], ibn, semis[y]).wait()
                pltpu.async_copy(h_hbm.at[ibn.at[0]], gbn, sems[y])

            pltpu.make_async_copy(h_hbm.at[ib.at[0]], gb, sems[x]).wait()
            pltpu.sync_copy(gb, agg_sh.at[ib.at[1]], add=True)
            if with_deg:
                pltpu.sync_copy(onesv, deg_sh.at[ib.at[1]], add=True)

            @pl.when(jnp.logical_not(last))
            def _():
                pltpu.async_copy(idx_hbm.at[wid, j + 2], ib, semis[x])

        def step(p, carry):
            j0 = 2 * p
            last = p + 1 >= C // 2
            half(j0, 0, last)
            half(j0 + 1, 1, last)
            return carry

        lax.fori_loop(0, C // 2, step, 0)
        plsc.subcore_barrier()
        pltpu.sync_copy(agg_sh.at[pl.ds(s * RPT, RPT)],
                        agg_out.at[c, pl.ds(s * RPT, RPT)])
        if with_deg:
            pltpu.sync_copy(deg_sh.at[pl.ds(s * RPT, RPT)],
                            deg_out.at[c, pl.ds(s * RPT, RPT)])

    return pl.kernel(body, out_type=out_type, mesh=_mesh,
                     scratch_types=scratch)


_sc_deg = _make_sc(True)
_sc_nodeg = _make_sc(False)

BM = 1000


def _tc_body(h_ref, agg_ref, deg_ref, Ws_ref, Wn_ref, b_ref, out_ref):
    inv = 1.0 / jnp.maximum(deg_ref[0] + deg_ref[1], 1.0)
    mean = (agg_ref[0] + agg_ref[1]) * inv
    acc = jnp.dot(h_ref[...], Ws_ref[...], preferred_element_type=jnp.float32)
    acc += jnp.dot(mean, Wn_ref[...], preferred_element_type=jnp.float32)
    acc += b_ref[...]
    out_ref[...] = jnp.maximum(acc, 0.0)


def _tc_dense(h, aggP, degP, Ws, Wn, b):
    return pl.pallas_call(
        _tc_body,
        grid=(N // BM,),
        in_specs=[
            pl.BlockSpec((BM, D), lambda i: (i, 0)),
            pl.BlockSpec((NC, BM, D), lambda i: (0, i, 0)),
            pl.BlockSpec((NC, BM, 1), lambda i: (0, i, 0)),
            pl.BlockSpec((D, D), lambda i: (0, 0)),
            pl.BlockSpec((D, D), lambda i: (0, 0)),
            pl.BlockSpec((1, D), lambda i: (0, 0)),
        ],
        out_specs=pl.BlockSpec((BM, D), lambda i: (i, 0)),
        out_shape=jax.ShapeDtypeStruct((N, D), jnp.float32),
    )(h, aggP, degP, Ws, Wn, b.reshape(1, D))


def kernel(x, edge_index, Ws1, Wn1, b1, Ws2, Wn2, b2):
    src = edge_index[0]
    dst = edge_index[1]
    pad = EP - E
    srcp = jnp.concatenate(
        [src, jnp.zeros((pad,), jnp.int32)]).reshape(NW, C, CH)
    dummy = N + (jnp.arange(pad, dtype=jnp.int32) % (NA - N))
    dstp = jnp.concatenate([dst, dummy]).reshape(NW, C, CH)
    idxp = jnp.stack([srcp, dstp], axis=2)
    zb = jnp.zeros((RPT, D), jnp.float32)
    zd = jnp.zeros((RPT,), jnp.float32)
    one = jnp.ones((CH,), jnp.float32)

    aggP1, degP = _sc_deg(x, idxp, zb, zd, one)
    degP = degP[:, :N, None]
    h1 = _tc_dense(x, aggP1[:, :N], degP, Ws1, Wn1, b1)
    aggP2 = _sc_nodeg(h1, idxp, zb, zd, one)
    if isinstance(aggP2, (list, tuple)):
        aggP2 = aggP2[0]
    h2 = _tc_dense(h1, aggP2[:, :N], degP, Ws2, Wn2, b2)
    return h2


# speedup vs baseline: 11.1594x; 2.8527x over previous
import functools

import jax
import jax.numpy as jnp
from jax import lax
from jax.experimental import pallas as pl
from jax.experimental.pallas import tpu as pltpu
from jax.experimental.pallas import tpu_sc as plsc

N = 10000
E = 320000
D = 128
NC = 2
NS = 16
NW = NC * NS
PERW = 10240
CH = 128
C = PERW // CH
EP = NW * PERW
NA = 10240
RPT = NA // NS

_mesh = plsc.VectorSubcoreMesh(core_axis_name="c", subcore_axis_name="s")


def _make_sc(with_deg):
    out_type = [jax.ShapeDtypeStruct((NC, NA, D), jnp.float32)]
    if with_deg:
        out_type.append(jax.ShapeDtypeStruct((NC, NA), jnp.float32))
    scratch = [
        pltpu.VMEM((2, CH), jnp.int32),
        pltpu.VMEM((2, CH), jnp.int32),
        pltpu.VMEM((CH, D), jnp.float32),
        pltpu.VMEM((CH, D), jnp.float32),
        pltpu.VMEM((CH,), jnp.float32),
        pltpu.MemorySpace.VMEM_SHARED((NA, D), jnp.float32),
        pltpu.MemorySpace.VMEM_SHARED((NA,), jnp.float32),
        pltpu.SemaphoreType.DMA,
        pltpu.SemaphoreType.DMA,
        pltpu.SemaphoreType.DMA,
        pltpu.SemaphoreType.DMA,
    ]

    def body(h_hbm, idx_hbm, zb_hbm, zd_hbm, one_hbm, *rest):
        if with_deg:
            agg_out, deg_out = rest[0], rest[1]
            scr = rest[2:]
        else:
            agg_out = rest[0]
            scr = rest[1:]
        (ibuf0, ibuf1, buf0, buf1, onesv, agg_sh, deg_sh,
         sem0, sem1, semi0, semi1) = scr
        ibufs, bufs = (ibuf0, ibuf1), (buf0, buf1)
        sems, semis = (sem0, sem1), (semi0, semi1)
        c = lax.axis_index("c")
        s = lax.axis_index("s")
        wid = c * NS + s
        pltpu.sync_copy(zb_hbm, agg_sh.at[pl.ds(s * RPT, RPT)])
        if with_deg:
            pltpu.sync_copy(zd_hbm, deg_sh.at[pl.ds(s * RPT, RPT)])
            pltpu.sync_copy(one_hbm, onesv)
        plsc.subcore_barrier()

        pltpu.sync_copy(idx_hbm.at[wid, 0], ibuf0)
        pltpu.async_copy(idx_hbm.at[wid, 1], ibuf1, semi1)
        pltpu.async_copy(h_hbm.at[ibuf0.at[0]], buf0, sem0)

        def half(j, x, last):
            y = 1 - x
            ib, gb = ibufs[x], bufs[x]
            ibn, gbn = ibufs[y], bufs[y]

            @pl.when(j + 1 < C)
            def _():
                pltpu.make_async_copy(
                    idx_hbm.at[wid, j + 1], ibn, semis[y]).wait()
                pltpu.async_copy(h_hbm.at[ibn.at[0]], gbn, sems[y])

            pltpu.make_async_copy(h_hbm.at[ib.at[0]], gb, sems[x]).wait()
            pltpu.sync_copy(gb, agg_sh.at[ib.at[1]], add=True)
            if with_deg:
                pltpu.sync_copy(onesv, deg_sh.at[ib.at[1]], add=True)

            @pl.when(jnp.logical_not(last))
            def _():
                pltpu.async_copy(idx_hbm.at[wid, j + 2], ib, semis[x])

        def step(p, carry):
            j0 = 2 * p
            last = p + 1 >= C // 2
            half(j0, 0, last)
            half(j0 + 1, 1, last)
            return carry

        lax.fori_loop(0, C // 2, step, 0)
        plsc.subcore_barrier()
        pltpu.sync_copy(agg_sh.at[pl.ds(s * RPT, RPT)],
                        agg_out.at[c, pl.ds(s * RPT, RPT)])
        if with_deg:
            pltpu.sync_copy(deg_sh.at[pl.ds(s * RPT, RPT)],
                            deg_out.at[c, pl.ds(s * RPT, RPT)])

    return pl.kernel(body, out_type=out_type, mesh=_mesh,
                     scratch_types=scratch)


_sc_deg = _make_sc(True)
_sc_nodeg = _make_sc(False)

BM = 1000


def _tc_body(h_ref, agg_ref, deg_ref, Ws_ref, Wn_ref, b_ref, out_ref):
    inv = 1.0 / jnp.maximum(deg_ref[0] + deg_ref[1], 1.0)
    mean = (agg_ref[0] + agg_ref[1]) * inv
    acc = jnp.dot(h_ref[...], Ws_ref[...], preferred_element_type=jnp.float32)
    acc += jnp.dot(mean, Wn_ref[...], preferred_element_type=jnp.float32)
    acc += b_ref[...]
    out_ref[...] = jnp.maximum(acc, 0.0)


def _tc_dense(h, aggP, degP, Ws, Wn, b):
    return pl.pallas_call(
        _tc_body,
        grid=(N // BM,),
        in_specs=[
            pl.BlockSpec((BM, D), lambda i: (i, 0)),
            pl.BlockSpec((NC, BM, D), lambda i: (0, i, 0)),
            pl.BlockSpec((NC, BM, 1), lambda i: (0, i, 0)),
            pl.BlockSpec((D, D), lambda i: (0, 0)),
            pl.BlockSpec((D, D), lambda i: (0, 0)),
            pl.BlockSpec((1, D), lambda i: (0, 0)),
        ],
        out_specs=pl.BlockSpec((BM, D), lambda i: (i, 0)),
        out_shape=jax.ShapeDtypeStruct((N, D), jnp.float32),
    )(h, aggP, degP, Ws, Wn, b.reshape(1, D))


def kernel(x, edge_index, Ws1, Wn1, b1, Ws2, Wn2, b2):
    src = edge_index[0]
    dst = edge_index[1]
    pw = PERW - E // NW
    pad_src = jnp.broadcast_to(jnp.arange(pw, dtype=jnp.int32), (NW, pw))
    pad_dst = jnp.broadcast_to(
        N + jnp.arange(pw, dtype=jnp.int32) % (NA - N), (NW, pw))
    srcp = jnp.concatenate(
        [src.reshape(NW, E // NW), pad_src], axis=1).reshape(NW, C, CH)
    dstp = jnp.concatenate(
        [dst.reshape(NW, E // NW), pad_dst], axis=1).reshape(NW, C, CH)
    idxp = jnp.stack([srcp, dstp], axis=2)
    zb = jnp.zeros((RPT, D), jnp.float32)
    zd = jnp.zeros((RPT,), jnp.float32)
    one = jnp.ones((CH,), jnp.float32)

    aggP1, degP = _sc_deg(x, idxp, zb, zd, one)
    degP = degP[:, :N, None]
    h1 = _tc_dense(x, aggP1[:, :N], degP, Ws1, Wn1, b1)
    aggP2 = _sc_nodeg(h1, idxp, zb, zd, one)
    if isinstance(aggP2, (list, tuple)):
        aggP2 = aggP2[0]
    h2 = _tc_dense(h1, aggP2[:, :N], degP, Ws2, Wn2, b2)
    return h2


# speedup vs baseline: 12.4498x; 1.1156x over previous
import jax
import jax.numpy as jnp
from jax import lax
from jax.experimental import pallas as pl
from jax.experimental.pallas import tpu as pltpu
from jax.experimental.pallas import tpu_sc as plsc

N = 10000
E = 320000
D = 128
NC = 2
NS = 16
NW = NC * NS
PW = E // NW
CH = 128
CF = PW // CH
TAIL = PW - CF * CH
NI = 4
NG = 2
NA = 10240
RPT = NA // NS

_mesh = plsc.VectorSubcoreMesh(core_axis_name="c", subcore_axis_name="s")


def _make_sc(with_deg):
    out_type = [jax.ShapeDtypeStruct((NC, NA, D), jnp.float32)]
    if with_deg:
        out_type.append(jax.ShapeDtypeStruct((NC, NA), jnp.float32))
    scratch = (
        [pltpu.VMEM((CH,), jnp.int32) for _ in range(NI)]
        + [pltpu.VMEM((CH,), jnp.int32) for _ in range(NI)]
        + [pltpu.VMEM((CH, D), jnp.float32) for _ in range(NG)]
        + [pltpu.VMEM((TAIL,), jnp.int32),
           pltpu.VMEM((TAIL,), jnp.int32),
           pltpu.VMEM((TAIL, D), jnp.float32),
           pltpu.VMEM((CH,), jnp.float32),
           pltpu.VMEM((TAIL,), jnp.float32),
           pltpu.MemorySpace.VMEM_SHARED((NA, D), jnp.float32),
           pltpu.MemorySpace.VMEM_SHARED((NA,), jnp.float32)]
        + [pltpu.SemaphoreType.DMA] * (NI + 2 * NG
                                       + (NG if with_deg else 0) + 1)
    )

    def body(h_hbm, src_hbm, dst_hbm, tsrc_hbm, tdst_hbm, zb_hbm, zd_hbm,
             one_hbm, *rest):
        if with_deg:
            agg_out, deg_out = rest[0], rest[1]
            scr = rest[2:]
        else:
            agg_out = rest[0]
            scr = rest[1:]
        sb = scr[0:NI]
        db = scr[NI:2 * NI]
        gb = scr[2 * NI:2 * NI + NG]
        ts, td, tg, onesv, onest, agg_sh, deg_sh = scr[2 * NI + NG:
                                                       2 * NI + NG + 7]
        sems = scr[2 * NI + NG + 7:]
        si = sems[0:NI]
        sg = sems[NI:NI + NG]
        ss = sems[NI + NG:NI + 2 * NG]
        sd = sems[NI + 2 * NG:NI + 3 * NG] if with_deg else None
        tsem = sems[-1]
        c = lax.axis_index("c")
        s = lax.axis_index("s")
        wid = c * NS + s
        srpt = pl.multiple_of(s * RPT, 128)

        pltpu.sync_copy(zb_hbm, agg_sh.at[pl.ds(srpt, RPT)])
        if with_deg:
            pltpu.sync_copy(zd_hbm, deg_sh.at[pl.ds(srpt, RPT)])
            pltpu.sync_copy(one_hbm, onesv)
            pltpu.sync_copy(one_hbm.at[pl.ds(0, TAIL)], onest)
        plsc.subcore_barrier()

        def idx_start(j, ki):
            pltpu.async_copy(src_hbm.at[wid, j, 0], sb[ki], si[ki])
            pltpu.async_copy(dst_hbm.at[wid, j, 0], db[ki], si[ki])

        def idx_wait(j, ki):
            pltpu.make_async_copy(src_hbm.at[wid, j, 0], sb[ki],
                                  si[ki]).wait()
            pltpu.make_async_copy(dst_hbm.at[wid, j, 0], db[ki],
                                  si[ki]).wait()

        def gat_start(ki, kg):
            pltpu.async_copy(h_hbm.at[sb[ki]], gb[kg], sg[kg])

        def gat_wait(ki, kg):
            pltpu.make_async_copy(h_hbm.at[sb[ki]], gb[kg], sg[kg]).wait()

        def scat_start(kg, ki):
            pltpu.async_copy(gb[kg], agg_sh.at[db[ki]], ss[kg], add=True)
            if with_deg:
                pltpu.async_copy(onesv, deg_sh.at[db[ki]], sd[kg], add=True)

        def scat_wait(kg, ki):
            pltpu.make_async_copy(gb[kg], agg_sh.at[db[ki]], ss[kg]).wait()
            if with_deg:
                pltpu.make_async_copy(onesv, deg_sh.at[db[ki]],
                                      sd[kg]).wait()

        def chunk_ops(j, jm, first=False, do_idx3=True, do_idx1=True):
            if not first:
                scat_wait((jm + 1) % NG, (jm + 3) % NI)
            if do_idx3:
                idx_start(j + 3, (jm + 3) % NI)
            if do_idx1:
                idx_wait(j + 1, (jm + 1) % NI)
                gat_start((jm + 1) % NI, (jm + 1) % NG)
            gat_wait(jm % NI, jm % NG)
            scat_start(jm % NG, jm % NI)

        idx_start(0, 0)
        idx_start(1, 1)
        idx_start(2, 2)
        idx_wait(0, 0)
        gat_start(0, 0)
        chunk_ops(0, 0, first=True)
        for j in (1, 2, 3):
            chunk_ops(j, j)

        def step(t, carry):
            for i in range(4):
                chunk_ops(t * 4 + i, i)
            return carry

        lax.fori_loop(1, (CF - 6) // 4, step, 0)

        for j in range(CF - 6, CF):
            chunk_ops(j, j % NI, do_idx3=j + 3 <= CF - 1,
                      do_idx1=j + 1 <= CF - 1)
        scat_wait((CF - 1) % NG, (CF - 1) % NI)

        pltpu.sync_copy(tsrc_hbm.at[wid, 0], ts)
        pltpu.sync_copy(tdst_hbm.at[wid, 0], td)
        pltpu.async_copy(h_hbm.at[ts], tg, tsem).wait()
        pltpu.sync_copy(tg, agg_sh.at[td], add=True)
        if with_deg:
            pltpu.sync_copy(onest, deg_sh.at[td], add=True)

        plsc.subcore_barrier()
        pltpu.sync_copy(agg_sh.at[pl.ds(srpt, RPT)],
                        agg_out.at[c, pl.ds(srpt, RPT)])
        if with_deg:
            pltpu.sync_copy(deg_sh.at[pl.ds(srpt, RPT)],
                            deg_out.at[c, pl.ds(srpt, RPT)])

    return pl.kernel(body, out_type=out_type, mesh=_mesh,
                     scratch_types=scratch)


_sc_deg = _make_sc(True)
_sc_nodeg = _make_sc(False)

BM = 1000


def _tc_body(h_ref, agg_ref, deg_ref, Ws_ref, Wn_ref, b_ref, out_ref):
    inv = 1.0 / jnp.maximum(deg_ref[0] + deg_ref[1], 1.0)
    mean = (agg_ref[0] + agg_ref[1]) * inv
    acc = jnp.dot(h_ref[...], Ws_ref[...], preferred_element_type=jnp.float32)
    acc += jnp.dot(mean, Wn_ref[...], preferred_element_type=jnp.float32)
    acc += b_ref[...]
    out_ref[...] = jnp.maximum(acc, 0.0)


def _tc_dense(h, aggP, degP, Ws, Wn, b):
    return pl.pallas_call(
        _tc_body,
        grid=(N // BM,),
        in_specs=[
            pl.BlockSpec((BM, D), lambda i: (i, 0)),
            pl.BlockSpec((NC, BM, D), lambda i: (0, i, 0)),
            pl.BlockSpec((NC, BM, 1), lambda i: (0, i, 0)),
            pl.BlockSpec((D, D), lambda i: (0, 0)),
            pl.BlockSpec((D, D), lambda i: (0, 0)),
            pl.BlockSpec((1, D), lambda i: (0, 0)),
        ],
        out_specs=pl.BlockSpec((BM, D), lambda i: (i, 0)),
        out_shape=jax.ShapeDtypeStruct((N, D), jnp.float32),
    )(h, aggP, degP, Ws, Wn, b.reshape(1, D))


def kernel(x, edge_index, Ws1, Wn1, b1, Ws2, Wn2, b2):
    src = edge_index[0]
    dst = edge_index[1]
    srcw = src.reshape(NW, PW)
    dstw = dst.reshape(NW, PW)
    bsrc = srcw[:, :CF * CH].reshape(NW, CF, 1, CH)
    bdst = dstw[:, :CF * CH].reshape(NW, CF, 1, CH)
    tsrc = srcw[:, CF * CH:].reshape(NW, 1, TAIL)
    tdst = dstw[:, CF * CH:].reshape(NW, 1, TAIL)
    zb = jnp.zeros((RPT, D), jnp.float32)
    zd = jnp.zeros((RPT,), jnp.float32)
    one = jnp.ones((CH,), jnp.float32)

    aggP1, degP = _sc_deg(x, bsrc, bdst, tsrc, tdst, zb, zd, one)
    degP = degP[:, :N, None]
    h1 = _tc_dense(x, aggP1[:, :N], degP, Ws1, Wn1, b1)
    aggP2 = _sc_nodeg(h1, bsrc, bdst, tsrc, tdst, zb, zd, one)
    if isinstance(aggP2, (list, tuple)):
        aggP2 = aggP2[0]
    h2 = _tc_dense(h1, aggP2[:, :N], degP, Ws2, Wn2, b2)
    return h2


# speedup vs baseline: 12.6324x; 1.0147x over previous
import jax
import jax.numpy as jnp
from jax import lax
from jax.experimental import pallas as pl
from jax.experimental.pallas import tpu as pltpu
from jax.experimental.pallas import tpu_sc as plsc

N = 10000
E = 320000
D = 128
NC = 2
NS = 16
NW = NC * NS
PW = E // NW
CH = 128
CF = PW // CH
TAIL = PW - CF * CH
NI = 4
NG = 2
NA = 10240
RPT = NA // NS

_mesh = plsc.VectorSubcoreMesh(core_axis_name="c", subcore_axis_name="s")


def _make_sc(with_deg):
    out_type = [jax.ShapeDtypeStruct((NC, NA, D), jnp.float32)]
    if with_deg:
        out_type.append(jax.ShapeDtypeStruct((NC, NA), jnp.float32))
    scratch = (
        [pltpu.VMEM((CH,), jnp.int32) for _ in range(NI)]
        + [pltpu.VMEM((CH,), jnp.int32) for _ in range(NI)]
        + [pltpu.VMEM((CH, D), jnp.float32) for _ in range(NG)]
        + [pltpu.VMEM((TAIL,), jnp.int32),
           pltpu.VMEM((TAIL,), jnp.int32),
           pltpu.VMEM((TAIL, D), jnp.float32),
           pltpu.VMEM((CH,), jnp.float32),
           pltpu.VMEM((TAIL,), jnp.float32),
           pltpu.MemorySpace.VMEM_SHARED((NA, D), jnp.float32),
           pltpu.MemorySpace.VMEM_SHARED((NA,), jnp.float32)]
        + [pltpu.SemaphoreType.DMA] * (NI + 2 * NG
                                       + (NG if with_deg else 0) + 1)
    )

    def body(h_hbm, src_hbm, dst_hbm, tsrc_hbm, tdst_hbm, zb_hbm, zd_hbm,
             one_hbm, *rest):
        if with_deg:
            agg_out, deg_out = rest[0], rest[1]
            scr = rest[2:]
        else:
            agg_out = rest[0]
            scr = rest[1:]
        sb = scr[0:NI]
        db = scr[NI:2 * NI]
        gb = scr[2 * NI:2 * NI + NG]
        ts, td, tg, onesv, onest, agg_sh, deg_sh = scr[2 * NI + NG:
                                                       2 * NI + NG + 7]
        sems = scr[2 * NI + NG + 7:]
        si = sems[0:NI]
        sg = sems[NI:NI + NG]
        ss = sems[NI + NG:NI + 2 * NG]
        sd = sems[NI + 2 * NG:NI + 3 * NG] if with_deg else None
        tsem = sems[-1]
        c = lax.axis_index("c")
        s = lax.axis_index("s")
        wid = c * NS + s
        srpt = pl.multiple_of(s * RPT, 128)

        pltpu.sync_copy(zb_hbm, agg_sh.at[pl.ds(srpt, RPT)])
        if with_deg:
            pltpu.sync_copy(zd_hbm, deg_sh.at[pl.ds(srpt, RPT)])
            pltpu.sync_copy(one_hbm, onesv)
            pltpu.sync_copy(one_hbm.at[pl.ds(0, TAIL)], onest)
        plsc.subcore_barrier()

        def idx_start(j, ki):
            pltpu.async_copy(src_hbm.at[wid, j, 0], sb[ki], si[ki])
            pltpu.async_copy(dst_hbm.at[wid, j, 0], db[ki], si[ki])

        def idx_wait(j, ki):
            pltpu.make_async_copy(src_hbm.at[wid, j, 0], sb[ki],
                                  si[ki]).wait()
            pltpu.make_async_copy(dst_hbm.at[wid, j, 0], db[ki],
                                  si[ki]).wait()

        def gat_start(ki, kg):
            pltpu.async_copy(h_hbm.at[sb[ki]], gb[kg], sg[kg])

        def gat_wait(ki, kg):
            pltpu.make_async_copy(h_hbm.at[sb[ki]], gb[kg], sg[kg]).wait()

        def scat_start(kg, ki):
            pltpu.async_copy(gb[kg], agg_sh.at[db[ki]], ss[kg], add=True)
            if with_deg:
                pltpu.async_copy(onesv, deg_sh.at[db[ki]], sd[kg], add=True)

        def scat_wait(kg, ki):
            pltpu.make_async_copy(gb[kg], agg_sh.at[db[ki]], ss[kg]).wait()
            if with_deg:
                pltpu.make_async_copy(onesv, deg_sh.at[db[ki]],
                                      sd[kg]).wait()

        def chunk_ops(j, jm, first=False, do_idx3=True, do_idx1=True):
            if not first:
                scat_wait((jm + 1) % NG, (jm + 3) % NI)
            if do_idx3:
                idx_start(j + 3, (jm + 3) % NI)
            if do_idx1:
                idx_wait(j + 1, (jm + 1) % NI)
                gat_start((jm + 1) % NI, (jm + 1) % NG)
            gat_wait(jm % NI, jm % NG)
            scat_start(jm % NG, jm % NI)

        idx_start(0, 0)
        idx_start(1, 1)
        idx_start(2, 2)
        idx_wait(0, 0)
        gat_start(0, 0)
        chunk_ops(0, 0, first=True)
        for j in (1, 2, 3):
            chunk_ops(j, j)

        def step(t, carry):
            for i in range(4):
                chunk_ops(t * 4 + i, i)
            return carry

        lax.fori_loop(1, (CF - 6) // 4, step, 0)

        for j in range(CF - 6, CF):
            chunk_ops(j, j % NI, do_idx3=j + 3 <= CF - 1,
                      do_idx1=j + 1 <= CF - 1)
        scat_wait((CF - 1) % NG, (CF - 1) % NI)

        pltpu.sync_copy(tsrc_hbm.at[wid, 0], ts)
        pltpu.sync_copy(tdst_hbm.at[wid, 0], td)
        pltpu.async_copy(h_hbm.at[ts], tg, tsem).wait()
        pltpu.sync_copy(tg, agg_sh.at[td], add=True)
        if with_deg:
            pltpu.sync_copy(onest, deg_sh.at[td], add=True)

        plsc.subcore_barrier()
        pltpu.sync_copy(agg_sh.at[pl.ds(srpt, RPT)],
                        agg_out.at[c, pl.ds(srpt, RPT)])
        if with_deg:
            pltpu.sync_copy(deg_sh.at[pl.ds(srpt, RPT)],
                            deg_out.at[c, pl.ds(srpt, RPT)])

    return pl.kernel(body, out_type=out_type, mesh=_mesh,
                     scratch_types=scratch)


_sc_deg = _make_sc(True)
_sc_nodeg = _make_sc(False)

BM = 1000


def _tc_self_body(h_ref, Ws_ref, b_ref, out_ref):
    out_ref[...] = jnp.dot(h_ref[...], Ws_ref[...],
                           preferred_element_type=jnp.float32) + b_ref[...]


def _tc_self(h, Ws, b):
    return pl.pallas_call(
        _tc_self_body,
        grid=(N // BM,),
        in_specs=[
            pl.BlockSpec((BM, D), lambda i: (i, 0)),
            pl.BlockSpec((D, D), lambda i: (0, 0)),
            pl.BlockSpec((1, D), lambda i: (0, 0)),
        ],
        out_specs=pl.BlockSpec((BM, D), lambda i: (i, 0)),
        out_shape=jax.ShapeDtypeStruct((N, D), jnp.float32),
    )(h, Ws, b.reshape(1, D))


def _tc_mean_body(tmp_ref, agg_ref, deg_ref, Wn_ref, out_ref):
    inv = 1.0 / jnp.maximum(deg_ref[0] + deg_ref[1], 1.0)
    mean = (agg_ref[0] + agg_ref[1]) * inv
    acc = tmp_ref[...] + jnp.dot(mean, Wn_ref[...],
                                 preferred_element_type=jnp.float32)
    out_ref[...] = jnp.maximum(acc, 0.0)


def _tc_mean(tmp, aggP, degP, Wn):
    return pl.pallas_call(
        _tc_mean_body,
        grid=(N // BM,),
        in_specs=[
            pl.BlockSpec((BM, D), lambda i: (i, 0)),
            pl.BlockSpec((NC, BM, D), lambda i: (0, i, 0)),
            pl.BlockSpec((NC, BM, 1), lambda i: (0, i, 0)),
            pl.BlockSpec((D, D), lambda i: (0, 0)),
        ],
        out_specs=pl.BlockSpec((BM, D), lambda i: (i, 0)),
        out_shape=jax.ShapeDtypeStruct((N, D), jnp.float32),
    )(tmp, aggP, degP, Wn)


def kernel(x, edge_index, Ws1, Wn1, b1, Ws2, Wn2, b2):
    src = edge_index[0]
    dst = edge_index[1]
    srcw = src.reshape(NW, PW)
    dstw = dst.reshape(NW, PW)
    bsrc = srcw[:, :CF * CH].reshape(NW, CF, 1, CH)
    bdst = dstw[:, :CF * CH].reshape(NW, CF, 1, CH)
    tsrc = srcw[:, CF * CH:].reshape(NW, 1, TAIL)
    tdst = dstw[:, CF * CH:].reshape(NW, 1, TAIL)
    zb = jnp.zeros((RPT, D), jnp.float32)
    zd = jnp.zeros((RPT,), jnp.float32)
    one = jnp.ones((CH,), jnp.float32)

    aggP1, degP = _sc_deg(x, bsrc, bdst, tsrc, tdst, zb, zd, one)
    degP = degP[:, :, None]
    tmp1 = _tc_self(x, Ws1, b1)
    h1 = _tc_mean(tmp1, aggP1, degP, Wn1)
    aggP2 = _sc_nodeg(h1, bsrc, bdst, tsrc, tdst, zb, zd, one)
    if isinstance(aggP2, (list, tuple)):
        aggP2 = aggP2[0]
    tmp2 = _tc_self(h1, Ws2, b2)
    h2 = _tc_mean(tmp2, aggP2, degP, Wn2)
    return h2


# speedup vs baseline: 12.7041x; 1.0057x over previous
import jax
import jax.numpy as jnp
import numpy as np
from jax import lax
from jax.experimental import pallas as pl
from jax.experimental.pallas import tpu as pltpu
from jax.experimental.pallas import tpu_sc as plsc

N = 10000
E = 320000
D = 128
NC = 2
NS = 16
NW = NC * NS
PW = E // NW
CH = 128
CF = PW // CH
TAIL = PW - CF * CH
NI = 4
NG = 2
NA = 10240
RPT = NA // NS

_mesh = plsc.VectorSubcoreMesh(core_axis_name="c", subcore_axis_name="s")


def _make_sc(with_deg):
    out_type = [jax.ShapeDtypeStruct((NC, NA, D), jnp.float32)]
    if with_deg:
        out_type.append(jax.ShapeDtypeStruct((NC, NA), jnp.float32))
    scratch = (
        [pltpu.VMEM((CH,), jnp.int32) for _ in range(NI)]
        + [pltpu.VMEM((CH,), jnp.int32) for _ in range(NI)]
        + [pltpu.VMEM((CH, D), jnp.float32) for _ in range(NG)]
        + [pltpu.VMEM((TAIL,), jnp.int32),
           pltpu.VMEM((TAIL,), jnp.int32),
           pltpu.VMEM((TAIL, D), jnp.float32),
           pltpu.VMEM((CH,), jnp.float32),
           pltpu.VMEM((TAIL,), jnp.float32),
           pltpu.MemorySpace.VMEM_SHARED((NA, D), jnp.float32),
           pltpu.MemorySpace.VMEM_SHARED((NA,), jnp.float32)]
        + [pltpu.SemaphoreType.DMA] * (NI + 2 * NG
                                       + (NG if with_deg else 0) + 1)
    )

    def body(h_hbm, src_hbm, dst_hbm, tsrc_hbm, tdst_hbm, zb_hbm, zd_hbm,
             one_hbm, *rest):
        if with_deg:
            agg_out, deg_out = rest[0], rest[1]
            scr = rest[2:]
        else:
            agg_out = rest[0]
            scr = rest[1:]
        sb = scr[0:NI]
        db = scr[NI:2 * NI]
        gb = scr[2 * NI:2 * NI + NG]
        ts, td, tg, onesv, onest, agg_sh, deg_sh = scr[2 * NI + NG:
                                                       2 * NI + NG + 7]
        sems = scr[2 * NI + NG + 7:]
        si = sems[0:NI]
        sg = sems[NI:NI + NG]
        ss = sems[NI + NG:NI + 2 * NG]
        sd = sems[NI + 2 * NG:NI + 3 * NG] if with_deg else None
        tsem = sems[-1]
        c = lax.axis_index("c")
        s = lax.axis_index("s")
        wid = c * NS + s
        srpt = pl.multiple_of(s * RPT, 128)

        def idx_start(j, ki):
            pltpu.async_copy(src_hbm.at[wid, j, 0], sb[ki], si[ki])
            pltpu.async_copy(dst_hbm.at[wid, j, 0], db[ki], si[ki])

        def idx_wait(j, ki):
            pltpu.make_async_copy(src_hbm.at[wid, j, 0], sb[ki],
                                  si[ki]).wait()
            pltpu.make_async_copy(dst_hbm.at[wid, j, 0], db[ki],
                                  si[ki]).wait()

        def gat_start(ki, kg):
            pltpu.async_copy(h_hbm.at[sb[ki]], gb[kg], sg[kg])

        def gat_wait(ki, kg):
            pltpu.make_async_copy(h_hbm.at[sb[ki]], gb[kg], sg[kg]).wait()

        def scat_start(kg, ki):
            pltpu.async_copy(gb[kg], agg_sh.at[db[ki]], ss[kg], add=True)
            if with_deg:
                pltpu.async_copy(onesv, deg_sh.at[db[ki]], sd[kg], add=True)

        def scat_wait(kg, ki):
            pltpu.make_async_copy(gb[kg], agg_sh.at[db[ki]], ss[kg]).wait()
            if with_deg:
                pltpu.make_async_copy(onesv, deg_sh.at[db[ki]],
                                      sd[kg]).wait()

        def chunk_ops(j, jm, first=False, do_idx3=True, do_idx1=True):
            if not first:
                scat_wait((jm + 1) % NG, (jm + 3) % NI)
            if do_idx3:
                idx_start(j + 3, (jm + 3) % NI)
            if do_idx1:
                idx_wait(j + 1, (jm + 1) % NI)
                gat_start((jm + 1) % NI, (jm + 1) % NG)
            gat_wait(jm % NI, jm % NG)
            scat_start(jm % NG, jm % NI)

        idx_start(0, 0)
        idx_start(1, 1)
        idx_start(2, 2)
        pltpu.sync_copy(zb_hbm, agg_sh.at[pl.ds(srpt, RPT)])
        if with_deg:
            pltpu.sync_copy(zd_hbm, deg_sh.at[pl.ds(srpt, RPT)])
            pltpu.sync_copy(one_hbm, onesv)
            pltpu.sync_copy(one_hbm.at[pl.ds(0, TAIL)], onest)
        idx_wait(0, 0)
        gat_start(0, 0)
        plsc.subcore_barrier()
        chunk_ops(0, 0, first=True)
        for j in (1, 2, 3):
            chunk_ops(j, j)

        def step(t, carry):
            for i in range(4):
                chunk_ops(t * 4 + i, i)
            return carry

        lax.fori_loop(1, (CF - 6) // 4, step, 0)

        for j in range(CF - 6, CF):
            chunk_ops(j, j % NI, do_idx3=j + 3 <= CF - 1,
                      do_idx1=j + 1 <= CF - 1)
        scat_wait((CF - 1) % NG, (CF - 1) % NI)

        pltpu.sync_copy(tsrc_hbm.at[wid, 0], ts)
        pltpu.sync_copy(tdst_hbm.at[wid, 0], td)
        pltpu.async_copy(h_hbm.at[ts], tg, tsem).wait()
        pltpu.sync_copy(tg, agg_sh.at[td], add=True)
        if with_deg:
            pltpu.sync_copy(onest, deg_sh.at[td], add=True)

        plsc.subcore_barrier()
        pltpu.sync_copy(agg_sh.at[pl.ds(srpt, RPT)],
                        agg_out.at[c, pl.ds(srpt, RPT)])
        if with_deg:
            pltpu.sync_copy(deg_sh.at[pl.ds(srpt, RPT)],
                            deg_out.at[c, pl.ds(srpt, RPT)])

    return pl.kernel(body, out_type=out_type, mesh=_mesh,
                     scratch_types=scratch)


_sc_deg = _make_sc(True)
_sc_nodeg = _make_sc(False)

BM = 1000


def _tc_self_body(h_ref, Ws_ref, b_ref, out_ref):
    out_ref[...] = jnp.dot(h_ref[...], Ws_ref[...],
                           preferred_element_type=jnp.float32) + b_ref[...]


def _tc_self(h, Ws, b):
    return pl.pallas_call(
        _tc_self_body,
        grid=(N // BM,),
        in_specs=[
            pl.BlockSpec((BM, D), lambda i: (i, 0)),
            pl.BlockSpec((D, D), lambda i: (0, 0)),
            pl.BlockSpec((1, D), lambda i: (0, 0)),
        ],
        out_specs=pl.BlockSpec((BM, D), lambda i: (i, 0)),
        out_shape=jax.ShapeDtypeStruct((N, D), jnp.float32),
    )(h, Ws, b.reshape(1, D))


def _tc_mean_body(tmp_ref, agg_ref, deg_ref, Wn_ref, out_ref):
    inv = 1.0 / jnp.maximum(deg_ref[0] + deg_ref[1], 1.0)
    mean = (agg_ref[0] + agg_ref[1]) * inv
    acc = tmp_ref[...] + jnp.dot(mean, Wn_ref[...],
                                 preferred_element_type=jnp.float32)
    out_ref[...] = jnp.maximum(acc, 0.0)


def _tc_mean(tmp, aggP, degP, Wn):
    return pl.pallas_call(
        _tc_mean_body,
        grid=(N // BM,),
        in_specs=[
            pl.BlockSpec((BM, D), lambda i: (i, 0)),
            pl.BlockSpec((NC, BM, D), lambda i: (0, i, 0)),
            pl.BlockSpec((NC, BM, 1), lambda i: (0, i, 0)),
            pl.BlockSpec((D, D), lambda i: (0, 0)),
        ],
        out_specs=pl.BlockSpec((BM, D), lambda i: (i, 0)),
        out_shape=jax.ShapeDtypeStruct((N, D), jnp.float32),
    )(tmp, aggP, degP, Wn)


_ZB = np.zeros((RPT, D), np.float32)
_ZD = np.zeros((RPT,), np.float32)
_ONE = np.ones((CH,), np.float32)


def kernel(x, edge_index, Ws1, Wn1, b1, Ws2, Wn2, b2):
    src = edge_index[0]
    dst = edge_index[1]
    srcw = src.reshape(NW, PW)
    dstw = dst.reshape(NW, PW)
    bsrc = srcw[:, :CF * CH].reshape(NW, CF, 1, CH)
    bdst = dstw[:, :CF * CH].reshape(NW, CF, 1, CH)
    tsrc = srcw[:, CF * CH:].reshape(NW, 1, TAIL)
    tdst = dstw[:, CF * CH:].reshape(NW, 1, TAIL)
    zb, zd, one = _ZB, _ZD, _ONE

    aggP1, degP = _sc_deg(x, bsrc, bdst, tsrc, tdst, zb, zd, one)
    degP = degP[:, :, None]
    tmp1 = _tc_self(x, Ws1, b1)
    h1 = _tc_mean(tmp1, aggP1, degP, Wn1)
    aggP2 = _sc_nodeg(h1, bsrc, bdst, tsrc, tdst, zb, zd, one)
    if isinstance(aggP2, (list, tuple)):
        aggP2 = aggP2[0]
    tmp2 = _tc_self(h1, Ws2, b2)
    h2 = _tc_mean(tmp2, aggP2, degP, Wn2)
    return h2


# speedup vs baseline: 13.1160x; 1.0324x over previous
import jax
import jax.numpy as jnp
import numpy as np
from jax import lax
from jax.experimental import pallas as pl
from jax.experimental.pallas import tpu as pltpu
from jax.experimental.pallas import tpu_sc as plsc

N = 10000
E = 320000
D = 128
NC = 2
NS = 16
NW = NC * NS
PW = E // NW
CH = 128
CF = PW // CH
TAIL = PW - CF * CH
NI = 4
NG = 2
NA = 10240
RPT = NA // NS

_mesh = plsc.VectorSubcoreMesh(core_axis_name="c", subcore_axis_name="s")


def _make_sc(with_deg):
    out_type = [jax.ShapeDtypeStruct((NC, NA, D), jnp.float32)]
    if with_deg:
        out_type.append(jax.ShapeDtypeStruct((NC, NA), jnp.float32))
    scratch = (
        [pltpu.VMEM((CH,), jnp.int32) for _ in range(NI)]
        + [pltpu.VMEM((CH,), jnp.int32) for _ in range(NI)]
        + [pltpu.VMEM((CH, D), jnp.float32) for _ in range(NG)]
        + [pltpu.VMEM((TAIL,), jnp.int32),
           pltpu.VMEM((TAIL,), jnp.int32),
           pltpu.VMEM((TAIL, D), jnp.float32),
           pltpu.VMEM((CH,), jnp.float32),
           pltpu.VMEM((TAIL,), jnp.float32),
           pltpu.MemorySpace.VMEM_SHARED((NA, D), jnp.float32),
           pltpu.MemorySpace.VMEM_SHARED((NA,), jnp.float32)]
        + [pltpu.SemaphoreType.DMA] * (NI + 2 * NG
                                       + (NG if with_deg else 0) + 1)
    )

    def body(h_hbm, src_hbm, dst_hbm, zb_hbm, zd_hbm, one_hbm, *rest):
        if with_deg:
            agg_out, deg_out = rest[0], rest[1]
            scr = rest[2:]
        else:
            agg_out = rest[0]
            scr = rest[1:]
        sb = scr[0:NI]
        db = scr[NI:2 * NI]
        gb = scr[2 * NI:2 * NI + NG]
        ts, td, tg, onesv, onest, agg_sh, deg_sh = scr[2 * NI + NG:
                                                       2 * NI + NG + 7]
        sems = scr[2 * NI + NG + 7:]
        si = sems[0:NI]
        sg = sems[NI:NI + NG]
        ss = sems[NI + NG:NI + 2 * NG]
        sd = sems[NI + 2 * NG:NI + 3 * NG] if with_deg else None
        tsem = sems[-1]
        c = lax.axis_index("c")
        s = lax.axis_index("s")
        wid = c * NS + s
        srpt = pl.multiple_of(s * RPT, 128)

        def _off(j):
            return pl.multiple_of(j * CH, CH)

        def idx_start(j, ki):
            pltpu.async_copy(src_hbm.at[wid, pl.ds(_off(j), CH)], sb[ki],
                             si[ki])
            pltpu.async_copy(dst_hbm.at[wid, pl.ds(_off(j), CH)], db[ki],
                             si[ki])

        def idx_wait(j, ki):
            pltpu.make_async_copy(src_hbm.at[wid, pl.ds(_off(j), CH)],
                                  sb[ki], si[ki]).wait()
            pltpu.make_async_copy(dst_hbm.at[wid, pl.ds(_off(j), CH)],
                                  db[ki], si[ki]).wait()

        def gat_start(ki, kg):
            pltpu.async_copy(h_hbm.at[sb[ki]], gb[kg], sg[kg])

        def gat_wait(ki, kg):
            pltpu.make_async_copy(h_hbm.at[sb[ki]], gb[kg], sg[kg]).wait()

        def scat_start(kg, ki):
            pltpu.async_copy(gb[kg], agg_sh.at[db[ki]], ss[kg], add=True)
            if with_deg:
                pltpu.async_copy(onesv, deg_sh.at[db[ki]], sd[kg], add=True)

        def scat_wait(kg, ki):
            pltpu.make_async_copy(gb[kg], agg_sh.at[db[ki]], ss[kg]).wait()
            if with_deg:
                pltpu.make_async_copy(onesv, deg_sh.at[db[ki]],
                                      sd[kg]).wait()

        def chunk_ops(j, jm, first=False, do_idx3=True, do_idx1=True):
            if not first:
                scat_wait((jm + 1) % NG, (jm + 3) % NI)
            if do_idx3:
                idx_start(j + 3, (jm + 3) % NI)
            if do_idx1:
                idx_wait(j + 1, (jm + 1) % NI)
                gat_start((jm + 1) % NI, (jm + 1) % NG)
            gat_wait(jm % NI, jm % NG)
            scat_start(jm % NG, jm % NI)

        idx_start(0, 0)
        idx_start(1, 1)
        idx_start(2, 2)
        pltpu.sync_copy(zb_hbm, agg_sh.at[pl.ds(srpt, RPT)])
        if with_deg:
            pltpu.sync_copy(zd_hbm, deg_sh.at[pl.ds(srpt, RPT)])
            pltpu.sync_copy(one_hbm, onesv)
            pltpu.sync_copy(one_hbm.at[pl.ds(0, TAIL)], onest)
        idx_wait(0, 0)
        gat_start(0, 0)
        plsc.subcore_barrier()
        chunk_ops(0, 0, first=True)
        for j in (1, 2, 3):
            chunk_ops(j, j)

        def step(t, carry):
            for i in range(4):
                chunk_ops(t * 4 + i, i)
            return carry

        lax.fori_loop(1, (CF - 6) // 4, step, 0)

        for j in range(CF - 6, CF):
            chunk_ops(j, j % NI, do_idx3=j + 3 <= CF - 1,
                      do_idx1=j + 1 <= CF - 1)
        scat_wait((CF - 1) % NG, (CF - 1) % NI)

        pltpu.sync_copy(src_hbm.at[wid, pl.ds(CF * CH, TAIL)], ts)
        pltpu.sync_copy(dst_hbm.at[wid, pl.ds(CF * CH, TAIL)], td)
        pltpu.async_copy(h_hbm.at[ts], tg, tsem).wait()
        pltpu.sync_copy(tg, agg_sh.at[td], add=True)
        if with_deg:
            pltpu.sync_copy(onest, deg_sh.at[td], add=True)

        plsc.subcore_barrier()
        pltpu.sync_copy(agg_sh.at[pl.ds(srpt, RPT)],
                        agg_out.at[c, pl.ds(srpt, RPT)])
        if with_deg:
            pltpu.sync_copy(deg_sh.at[pl.ds(srpt, RPT)],
                            deg_out.at[c, pl.ds(srpt, RPT)])

    return pl.kernel(body, out_type=out_type, mesh=_mesh,
                     scratch_types=scratch)


_sc_deg = _make_sc(True)
_sc_nodeg = _make_sc(False)

BM = 1000


def _tc_self_body(h_ref, Ws_ref, b_ref, out_ref):
    out_ref[...] = jnp.dot(h_ref[...], Ws_ref[...],
                           preferred_element_type=jnp.float32) + b_ref[...]


def _tc_self(h, Ws, b):
    return pl.pallas_call(
        _tc_self_body,
        grid=(N // BM,),
        in_specs=[
            pl.BlockSpec((BM, D), lambda i: (i, 0)),
            pl.BlockSpec((D, D), lambda i: (0, 0)),
            pl.BlockSpec((1, D), lambda i: (0, 0)),
        ],
        out_specs=pl.BlockSpec((BM, D), lambda i: (i, 0)),
        out_shape=jax.ShapeDtypeStruct((N, D), jnp.float32),
    )(h, Ws, b.reshape(1, D))


def _tc_mean_body(tmp_ref, agg_ref, deg_ref, Wn_ref, out_ref):
    inv = 1.0 / jnp.maximum(deg_ref[0] + deg_ref[1], 1.0)
    mean = (agg_ref[0] + agg_ref[1]) * inv
    acc = tmp_ref[...] + jnp.dot(mean, Wn_ref[...],
                                 preferred_element_type=jnp.float32)
    out_ref[...] = jnp.maximum(acc, 0.0)


def _tc_mean(tmp, aggP, degP, Wn):
    return pl.pallas_call(
        _tc_mean_body,
        grid=(N // BM,),
        in_specs=[
            pl.BlockSpec((BM, D), lambda i: (i, 0)),
            pl.BlockSpec((NC, BM, D), lambda i: (0, i, 0)),
            pl.BlockSpec((NC, BM, 1), lambda i: (0, i, 0)),
            pl.BlockSpec((D, D), lambda i: (0, 0)),
        ],
        out_specs=pl.BlockSpec((BM, D), lambda i: (i, 0)),
        out_shape=jax.ShapeDtypeStruct((N, D), jnp.float32),
    )(tmp, aggP, degP, Wn)


_ZB = np.zeros((RPT, D), np.float32)
_ZD = np.zeros((RPT,), np.float32)
_ONE = np.ones((CH,), np.float32)


def kernel(x, edge_index, Ws1, Wn1, b1, Ws2, Wn2, b2):
    src = edge_index[0]
    dst = edge_index[1]
    srcw = src.reshape(NW, PW)
    dstw = dst.reshape(NW, PW)
    zb, zd, one = _ZB, _ZD, _ONE

    aggP1, degP = _sc_deg(x, srcw, dstw, zb, zd, one)
    degP = degP[:, :, None]
    tmp1 = _tc_self(x, Ws1, b1)
    h1 = _tc_mean(tmp1, aggP1, degP, Wn1)
    aggP2 = _sc_nodeg(h1, srcw, dstw, zb, zd, one)
    if isinstance(aggP2, (list, tuple)):
        aggP2 = aggP2[0]
    tmp2 = _tc_self(h1, Ws2, b2)
    h2 = _tc_mean(tmp2, aggP2, degP, Wn2)
    return h2


# speedup vs baseline: 13.1885x; 1.0055x over previous
import jax
import jax.numpy as jnp
import numpy as np
from jax import lax
from jax.experimental import pallas as pl
from jax.experimental.pallas import tpu as pltpu
from jax.experimental.pallas import tpu_sc as plsc

N = 10000
E = 320000
D = 128
NC = 2
NS = 16
NW = NC * NS
PW = E // NW
CH = 128
CF = 79
PWP = CF * CH
NI = 4
NG = 2
NA = 10240
RPT = NA // NS

_mesh = plsc.VectorSubcoreMesh(core_axis_name="c", subcore_axis_name="s")


def _make_sc(with_deg):
    out_type = [jax.ShapeDtypeStruct((NC, NA, D), jnp.float32)]
    if with_deg:
        out_type.append(jax.ShapeDtypeStruct((NC, NA), jnp.float32))
    scratch = (
        [pltpu.VMEM((CH,), jnp.int32) for _ in range(NI)]
        + [pltpu.VMEM((CH,), jnp.int32) for _ in range(NI)]
        + [pltpu.VMEM((CH, D), jnp.float32) for _ in range(NG)]
        + [pltpu.VMEM((CH,), jnp.float32),
           pltpu.MemorySpace.VMEM_SHARED((NA, D), jnp.float32),
           pltpu.MemorySpace.VMEM_SHARED((NA,), jnp.float32)]
        + [pltpu.SemaphoreType.DMA] * (NI + 2 * NG
                                       + (NG if with_deg else 0))
    )

    def body(h_hbm, src_hbm, dst_hbm, zb_hbm, zd_hbm, one_hbm, *rest):
        if with_deg:
            agg_out, deg_out = rest[0], rest[1]
            scr = rest[2:]
        else:
            agg_out = rest[0]
            scr = rest[1:]
        sb = scr[0:NI]
        db = scr[NI:2 * NI]
        gb = scr[2 * NI:2 * NI + NG]
        onesv, agg_sh, deg_sh = scr[2 * NI + NG:2 * NI + NG + 3]
        sems = scr[2 * NI + NG + 3:]
        si = sems[0:NI]
        sg = sems[NI:NI + NG]
        ss = sems[NI + NG:NI + 2 * NG]
        sd = sems[NI + 2 * NG:NI + 3 * NG] if with_deg else None
        c = lax.axis_index("c")
        s = lax.axis_index("s")
        wid = c * NS + s
        srpt = pl.multiple_of(s * RPT, 128)

        def _off(j):
            return pl.multiple_of(j * CH, CH)

        def idx_start(j, ki):
            pltpu.async_copy(src_hbm.at[wid, pl.ds(_off(j), CH)], sb[ki],
                             si[ki])
            pltpu.async_copy(dst_hbm.at[wid, pl.ds(_off(j), CH)], db[ki],
                             si[ki])

        def idx_wait(j, ki):
            pltpu.make_async_copy(src_hbm.at[wid, pl.ds(_off(j), CH)],
                                  sb[ki], si[ki]).wait()
            pltpu.make_async_copy(dst_hbm.at[wid, pl.ds(_off(j), CH)],
                                  db[ki], si[ki]).wait()

        def gat_start(ki, kg):
            pltpu.async_copy(h_hbm.at[sb[ki]], gb[kg], sg[kg])

        def gat_wait(ki, kg):
            pltpu.make_async_copy(h_hbm.at[sb[ki]], gb[kg], sg[kg]).wait()

        def scat_start(kg, ki):
            pltpu.async_copy(gb[kg], agg_sh.at[db[ki]], ss[kg], add=True)
            if with_deg:
                pltpu.async_copy(onesv, deg_sh.at[db[ki]], sd[kg], add=True)

        def scat_wait(kg, ki):
            pltpu.make_async_copy(gb[kg], agg_sh.at[db[ki]], ss[kg]).wait()
            if with_deg:
                pltpu.make_async_copy(onesv, deg_sh.at[db[ki]],
                                      sd[kg]).wait()

        def chunk_ops(j, jm, first=False, do_idx3=True, do_idx1=True):
            if not first:
                scat_wait((jm + 1) % NG, (jm + 3) % NI)
            if do_idx3:
                idx_start(j + 3, (jm + 3) % NI)
            if do_idx1:
                idx_wait(j + 1, (jm + 1) % NI)
                gat_start((jm + 1) % NI, (jm + 1) % NG)
            gat_wait(jm % NI, jm % NG)
            scat_start(jm % NG, jm % NI)

        idx_start(0, 0)
        idx_start(1, 1)
        idx_start(2, 2)
        pltpu.sync_copy(zb_hbm, agg_sh.at[pl.ds(srpt, RPT)])
        if with_deg:
            pltpu.sync_copy(zd_hbm, deg_sh.at[pl.ds(srpt, RPT)])
            pltpu.sync_copy(one_hbm, onesv)
        idx_wait(0, 0)
        gat_start(0, 0)
        plsc.subcore_barrier()
        chunk_ops(0, 0, first=True)
        for j in (1, 2, 3):
            chunk_ops(j, j)

        def step(t, carry):
            for i in range(4):
                chunk_ops(t * 4 + i, i)
            return carry

        lax.fori_loop(1, (CF - 6) // 4, step, 0)

        for j in range(72, CF):
            chunk_ops(j, j % NI, do_idx3=j + 3 <= CF - 1,
                      do_idx1=j + 1 <= CF - 1)
        scat_wait((CF - 1) % NG, (CF - 1) % NI)

        plsc.subcore_barrier()
        pltpu.sync_copy(agg_sh.at[pl.ds(srpt, RPT)],
                        agg_out.at[c, pl.ds(srpt, RPT)])
        if with_deg:
            pltpu.sync_copy(deg_sh.at[pl.ds(srpt, RPT)],
                            deg_out.at[c, pl.ds(srpt, RPT)])

    return pl.kernel(body, out_type=out_type, mesh=_mesh,
                     scratch_types=scratch)


_sc_deg = _make_sc(True)
_sc_nodeg = _make_sc(False)

BM = 1000


def _tc_self_body(h_ref, Ws_ref, b_ref, out_ref):
    out_ref[...] = jnp.dot(h_ref[...], Ws_ref[...],
                           preferred_element_type=jnp.float32) + b_ref[...]


def _tc_self(h, Ws, b):
    return pl.pallas_call(
        _tc_self_body,
        grid=(N // BM,),
        in_specs=[
            pl.BlockSpec((BM, D), lambda i: (i, 0)),
            pl.BlockSpec((D, D), lambda i: (0, 0)),
            pl.BlockSpec((1, D), lambda i: (0, 0)),
        ],
        out_specs=pl.BlockSpec((BM, D), lambda i: (i, 0)),
        out_shape=jax.ShapeDtypeStruct((N, D), jnp.float32),
    )(h, Ws, b.reshape(1, D))


def _tc_mean_body(tmp_ref, agg_ref, deg_ref, Wn_ref, out_ref):
    inv = 1.0 / jnp.maximum(deg_ref[0] + deg_ref[1], 1.0)
    mean = (agg_ref[0] + agg_ref[1]) * inv
    acc = tmp_ref[...] + jnp.dot(mean, Wn_ref[...],
                                 preferred_element_type=jnp.float32)
    out_ref[...] = jnp.maximum(acc, 0.0)


def _tc_mean(tmp, aggP, degP, Wn):
    return pl.pallas_call(
        _tc_mean_body,
        grid=(N // BM,),
        in_specs=[
            pl.BlockSpec((BM, D), lambda i: (i, 0)),
            pl.BlockSpec((NC, BM, D), lambda i: (0, i, 0)),
            pl.BlockSpec((NC, BM, 1), lambda i: (0, i, 0)),
            pl.BlockSpec((D, D), lambda i: (0, 0)),
        ],
        out_specs=pl.BlockSpec((BM, D), lambda i: (i, 0)),
        out_shape=jax.ShapeDtypeStruct((N, D), jnp.float32),
    )(tmp, aggP, degP, Wn)


_ZB = np.zeros((RPT, D), np.float32)
_ZD = np.zeros((RPT,), np.float32)
_ONE = np.ones((CH,), np.float32)


_PS = np.broadcast_to(np.arange(PWP - PW, dtype=np.int32), (NW, PWP - PW))
_PD = np.broadcast_to(N + np.arange(PWP - PW, dtype=np.int32),
                      (NW, PWP - PW))


def kernel(x, edge_index, Ws1, Wn1, b1, Ws2, Wn2, b2):
    src = edge_index[0]
    dst = edge_index[1]
    srcw = jnp.concatenate([src.reshape(NW, PW), _PS], axis=1)
    dstw = jnp.concatenate([dst.reshape(NW, PW), _PD], axis=1)
    zb, zd, one = _ZB, _ZD, _ONE

    aggP1, degP = _sc_deg(x, srcw, dstw, zb, zd, one)
    degP = degP[:, :, None]
    tmp1 = _tc_self(x, Ws1, b1)
    h1 = _tc_mean(tmp1, aggP1, degP, Wn1)
    aggP2 = _sc_nodeg(h1, srcw, dstw, zb, zd, one)
    if isinstance(aggP2, (list, tuple)):
        aggP2 = aggP2[0]
    tmp2 = _tc_self(h1, Ws2, b2)
    h2 = _tc_mean(tmp2, aggP2, degP, Wn2)
    return h2
